# Initial kernel scaffold; baseline (speedup 1.0000x reference)
import jax
import jax.numpy as jnp
from jax.experimental import pallas as pl


def kernel(x_ligand, x_pocket, edge_attr_l2p, edge_attr_p2l, W_lp_src, b_lp_src, W_lp_dst, b_lp_dst, W_lp_edge, b_lp_edge, w_fc_lp, b_fc_lp, W_pl_src, b_pl_src, W_pl_dst, b_pl_dst, W_pl_edge, b_pl_edge, w_fc_pl, b_fc_pl, edge_index_l2p, edge_index_p2l):
    raise NotImplementedError("write your pallas kernel here")


# speedup vs baseline: 2.0309x; 2.0309x over previous
import functools

import jax
import jax.numpy as jnp
from jax import lax
from jax.experimental import pallas as pl
from jax.experimental.pallas import tpu as pltpu
from jax.experimental.pallas import tpu_sc as plsc

N_NODE = 10000
E = 320000
D_NODE = 128
D_EDGE = 16
D_HID = 128

NC = 2
NS = 16
NW = NC * NS
EPW = E // NW
G = 80
NCHUNK = EPW // G


_RB = 2000


def _proj_body(xl_ref, xp_ref, wl_ref, wp_ref, bl_ref, bp_ref,
               hs_lp_ref, hd_pl_ref, hd_lp_ref, hs_pl_ref):
    yl = jnp.dot(xl_ref[...], wl_ref[...], preferred_element_type=jnp.float32, precision=lax.Precision.HIGHEST)
    yl = yl + bl_ref[...]
    hs_lp_ref[...] = yl[:, :D_HID]
    hd_pl_ref[...] = yl[:, D_HID:]
    yp = jnp.dot(xp_ref[...], wp_ref[...], preferred_element_type=jnp.float32, precision=lax.Precision.HIGHEST)
    yp = yp + bp_ref[...]
    hd_lp_ref[...] = yp[:, :D_HID]
    hs_pl_ref[...] = yp[:, D_HID:]


def _proj(xl, xp, wl, wp, bl, bp):
    n_blk = N_NODE // _RB
    f = jnp.float32
    return pl.pallas_call(
        _proj_body,
        grid=(n_blk,),
        in_specs=[
            pl.BlockSpec((_RB, D_NODE), lambda i: (i, 0)),
            pl.BlockSpec((_RB, D_NODE), lambda i: (i, 0)),
            pl.BlockSpec((D_NODE, 2 * D_HID), lambda i: (0, 0)),
            pl.BlockSpec((D_NODE, 2 * D_HID), lambda i: (0, 0)),
            pl.BlockSpec((1, 2 * D_HID), lambda i: (0, 0)),
            pl.BlockSpec((1, 2 * D_HID), lambda i: (0, 0)),
        ],
        out_specs=[pl.BlockSpec((_RB, D_HID), lambda i: (i, 0))] * 4,
        out_shape=[jax.ShapeDtypeStruct((N_NODE, D_HID), f)] * 4,
    )(xl, xp, wl, wp, bl, bp)


_SC_MESH = plsc.VectorSubcoreMesh(core_axis_name="c", subcore_axis_name="s")


@functools.partial(
    pl.kernel,
    out_type=[jax.ShapeDtypeStruct((E, D_HID), jnp.float32)] * 2,
    mesh=_SC_MESH,
    scratch_types=[
        pltpu.VMEM((G,), jnp.int32),
        pltpu.VMEM((G,), jnp.int32),
        pltpu.VMEM((G, D_HID), jnp.float32),
        pltpu.VMEM((G, D_HID), jnp.float32),
        pltpu.SemaphoreType.DMA,
        pltpu.SemaphoreType.DMA,
    ],
)
def _sc_gather_mul(hs_lp, hd_lp, src_lp, dst_lp, hs_pl, hd_pl, src_pl, dst_pl,
                   p_lp, p_pl, idx_s, idx_d, rows_s, rows_d, sem_a, sem_b):
    wid = lax.axis_index("s") * NC + lax.axis_index("c")
    base = wid * EPW

    def one_branch(hs, hd, srcv, dstv, pout):
        def chunk(i, carry):
            off = base + i * G
            pltpu.sync_copy(srcv.at[pl.ds(off, G)], idx_s)
            pltpu.sync_copy(dstv.at[pl.ds(off, G)], idx_d)
            ca = pltpu.async_copy(hs.at[idx_s], rows_s, sem_a)
            cb = pltpu.async_copy(hd.at[idx_d], rows_d, sem_b)
            ca.wait()
            cb.wait()

            def mulrow(j, c2):
                for k in range(D_HID // 16):
                    sl = pl.ds(k * 16, 16)
                    rows_s[j, sl] = rows_s[j, sl] * rows_d[j, sl]
                return c2

            lax.fori_loop(0, G, mulrow, 0, unroll=2)
            pltpu.sync_copy(rows_s, pout.at[pl.ds(off, G)])
            return carry

        lax.fori_loop(0, NCHUNK, chunk, 0)

    one_branch(hs_lp, hd_lp, src_lp, dst_lp, p_lp)
    one_branch(hs_pl, hd_pl, src_pl, dst_pl, p_pl)


_EB = 2560
_NEB = E // _EB


def _q_body(ea_lp_ref, p_lp_ref, ea_pl_ref, p_pl_ref, wc_lp_ref, wc_pl_ref,
            out_ref, q_lp, qb_lp, q_pl, qb_pl):
    i = pl.program_id(0)

    @pl.when(i == 0)
    def _init():
        q_lp[...] = jnp.zeros_like(q_lp)
        qb_lp[...] = jnp.zeros_like(qb_lp)
        q_pl[...] = jnp.zeros_like(q_pl)
        qb_pl[...] = jnp.zeros_like(qb_pl)

    q_lp[...] += jnp.dot(ea_lp_ref[...], p_lp_ref[...],
                         preferred_element_type=jnp.float32, precision=lax.Precision.HIGHEST)
    qb_lp[...] += jnp.sum(p_lp_ref[...], axis=0, keepdims=True)
    q_pl[...] += jnp.dot(ea_pl_ref[...], p_pl_ref[...],
                         preferred_element_type=jnp.float32, precision=lax.Precision.HIGHEST)
    qb_pl[...] += jnp.sum(p_pl_ref[...], axis=0, keepdims=True)

    @pl.when(i == _NEB - 1)
    def _fin():
        wc_lp = wc_lp_ref[...]
        wc_pl = wc_pl_ref[...]
        t_lp = (jnp.sum(wc_lp[:D_EDGE] * q_lp[...])
                + jnp.sum(wc_lp[D_EDGE:D_EDGE + 1] * qb_lp[...])
                + jnp.sum(wc_lp[D_EDGE + 1:D_EDGE + 2]))
        t_pl = (jnp.sum(wc_pl[:D_EDGE] * q_pl[...])
                + jnp.sum(wc_pl[D_EDGE:D_EDGE + 1] * qb_pl[...])
                + jnp.sum(wc_pl[D_EDGE + 1:D_EDGE + 2]))
        col = lax.broadcasted_iota(jnp.int32, (1, 2), 1)
        out_ref[...] = jnp.where(col == 0, t_lp, t_pl)


def _q_reduce(ea_t_lp, p_lp, ea_t_pl, p_pl, wc_lp, wc_pl):
    f = jnp.float32
    return pl.pallas_call(
        _q_body,
        grid=(_NEB,),
        in_specs=[
            pl.BlockSpec((D_EDGE, _EB), lambda i: (0, i)),
            pl.BlockSpec((_EB, D_HID), lambda i: (i, 0)),
            pl.BlockSpec((D_EDGE, _EB), lambda i: (0, i)),
            pl.BlockSpec((_EB, D_HID), lambda i: (i, 0)),
            pl.BlockSpec((24, D_HID), lambda i: (0, 0)),
            pl.BlockSpec((24, D_HID), lambda i: (0, 0)),
        ],
        out_specs=pl.BlockSpec((1, 2), lambda i: (0, 0)),
        out_shape=jax.ShapeDtypeStruct((1, 2), f),
        scratch_shapes=[
            pltpu.VMEM((D_EDGE, D_HID), f),
            pltpu.VMEM((1, D_HID), f),
            pltpu.VMEM((D_EDGE, D_HID), f),
            pltpu.VMEM((1, D_HID), f),
        ],
    )(ea_t_lp, p_lp, ea_t_pl, p_pl, wc_lp, wc_pl)


def kernel(x_ligand, x_pocket, edge_attr_l2p, edge_attr_p2l,
           W_lp_src, b_lp_src, W_lp_dst, b_lp_dst, W_lp_edge, b_lp_edge,
           w_fc_lp, b_fc_lp,
           W_pl_src, b_pl_src, W_pl_dst, b_pl_dst, W_pl_edge, b_pl_edge,
           w_fc_pl, b_fc_pl,
           edge_index_l2p, edge_index_p2l):
    f = jnp.float32

    wl = jnp.concatenate([W_lp_src, W_pl_dst], axis=1)
    wp = jnp.concatenate([W_lp_dst, W_pl_src], axis=1)
    bl = jnp.concatenate([b_lp_src, b_pl_dst])[None, :]
    bp = jnp.concatenate([b_lp_dst, b_pl_src])[None, :]

    src_lp = edge_index_l2p[0].astype(jnp.int32)
    dst_lp = edge_index_l2p[1].astype(jnp.int32)
    src_pl = edge_index_p2l[0].astype(jnp.int32)
    dst_pl = edge_index_p2l[1].astype(jnp.int32)

    ea_t_lp = edge_attr_l2p.T
    ea_t_pl = edge_attr_p2l.T

    def wcomb(We, be, wf, bf):
        wfv = wf[:, 0]
        rows = [We * wfv[None, :],
                (be * wfv)[None, :],
                jnp.full((1, D_HID), bf[0] * (E / float(D_HID)), dtype=f),
                jnp.zeros((24 - D_EDGE - 2, D_HID), dtype=f)]
        return jnp.concatenate(rows, axis=0)

    wc_lp = wcomb(W_lp_edge, b_lp_edge, w_fc_lp, b_fc_lp)
    wc_pl = wcomb(W_pl_edge, b_pl_edge, w_fc_pl, b_fc_pl)

    hs_lp, hd_pl, hd_lp, hs_pl = _proj(x_ligand, x_pocket, wl, wp, bl, bp)

    p_lp, p_pl = _sc_gather_mul(hs_lp, hd_lp, src_lp, dst_lp,
                                hs_pl, hd_pl, src_pl, dst_pl)

    return _q_reduce(ea_t_lp, p_lp, ea_t_pl, p_pl, wc_lp, wc_pl)


# speedup vs baseline: 3.0026x; 1.4785x over previous
import functools

import jax
import jax.numpy as jnp
from jax import lax
from jax.experimental import pallas as pl
from jax.experimental.pallas import tpu as pltpu
from jax.experimental.pallas import tpu_sc as plsc

N_NODE = 10000
E = 320000
D_NODE = 128
D_EDGE = 16
D_HID = 128

NC = 2
NS = 16
NW = NC * NS
EPW = E // NW
G = 80
NCHUNK = EPW // G


_RB = 2000


def _proj_body(xl_ref, xp_ref, wl_ref, wp_ref, bl_ref, bp_ref,
               hs_lp_ref, hd_pl_ref, hd_lp_ref, hs_pl_ref):
    yl = jnp.dot(xl_ref[...], wl_ref[...], preferred_element_type=jnp.float32, precision=lax.Precision.HIGHEST)
    yl = yl + bl_ref[...]
    hs_lp_ref[...] = yl[:, :D_HID]
    hd_pl_ref[...] = yl[:, D_HID:]
    yp = jnp.dot(xp_ref[...], wp_ref[...], preferred_element_type=jnp.float32, precision=lax.Precision.HIGHEST)
    yp = yp + bp_ref[...]
    hd_lp_ref[...] = yp[:, :D_HID]
    hs_pl_ref[...] = yp[:, D_HID:]


def _proj(xl, xp, wl, wp, bl, bp):
    n_blk = N_NODE // _RB
    f = jnp.float32
    return pl.pallas_call(
        _proj_body,
        grid=(n_blk,),
        in_specs=[
            pl.BlockSpec((_RB, D_NODE), lambda i: (i, 0)),
            pl.BlockSpec((_RB, D_NODE), lambda i: (i, 0)),
            pl.BlockSpec((D_NODE, 2 * D_HID), lambda i: (0, 0)),
            pl.BlockSpec((D_NODE, 2 * D_HID), lambda i: (0, 0)),
            pl.BlockSpec((1, 2 * D_HID), lambda i: (0, 0)),
            pl.BlockSpec((1, 2 * D_HID), lambda i: (0, 0)),
        ],
        out_specs=[pl.BlockSpec((_RB, D_HID), lambda i: (i, 0))] * 4,
        out_shape=[jax.ShapeDtypeStruct((N_NODE, D_HID), f)] * 4,
    )(xl, xp, wl, wp, bl, bp)


_SC_MESH = plsc.VectorSubcoreMesh(core_axis_name="c", subcore_axis_name="s")
NCHUNK2 = 2 * NCHUNK


@functools.partial(
    pl.kernel,
    out_type=jax.ShapeDtypeStruct((2 * E, D_HID), jnp.float32),
    mesh=_SC_MESH,
    scratch_types=[
        pltpu.VMEM((NCHUNK2, G), jnp.int32),
        pltpu.VMEM((NCHUNK2, G), jnp.int32),
        pltpu.VMEM((G, D_HID), jnp.float32),
        pltpu.VMEM((G, D_HID), jnp.float32),
        pltpu.VMEM((G, D_HID), jnp.float32),
        pltpu.VMEM((G, D_HID), jnp.float32),
        pltpu.VMEM((G, D_HID), jnp.float32),
        pltpu.VMEM((G, D_HID), jnp.float32),
        pltpu.SemaphoreType.DMA,
        pltpu.SemaphoreType.DMA,
        pltpu.SemaphoreType.DMA,
        pltpu.SemaphoreType.DMA,
        pltpu.SemaphoreType.DMA,
        pltpu.SemaphoreType.DMA,
    ],
)
def _sc_gather_mul(hs_full, hd_full, srcc, dstc, pout,
                   idx_s, idx_d, rs0, rs1, rd0, rd1, pr0, pr1,
                   ga0, ga1, gb0, gb1, ss0, ss1):
    wid = lax.axis_index("s") * NC + lax.axis_index("c")
    base = wid * EPW
    rows_s = (rs0, rs1)
    rows_d = (rd0, rd1)
    prod = (pr0, pr1)
    sem_a = (ga0, ga1)
    sem_b = (gb0, gb1)
    sem_s = (ss0, ss1)

    pltpu.sync_copy(srcc.at[wid], idx_s)
    pltpu.sync_copy(dstc.at[wid], idx_d)

    def p_off(t):
        return base + t * G + jnp.where(t >= NCHUNK, E - EPW * NW, 0)

    def start_gather(t, b):
        pltpu.async_copy(hs_full.at[idx_s.at[t]], rows_s[b], sem_a[b])
        pltpu.async_copy(hd_full.at[idx_d.at[t]], rows_d[b], sem_b[b])

    def wait_gather(t, b):
        pltpu.make_async_copy(hs_full.at[idx_s.at[t]], rows_s[b], sem_a[b]).wait()
        pltpu.make_async_copy(hd_full.at[idx_d.at[t]], rows_d[b], sem_b[b]).wait()

    def start_store(t, b):
        pltpu.async_copy(prod[b], pout.at[pl.ds(p_off(t), G)], sem_s[b])

    def wait_store(t, b):
        pltpu.make_async_copy(prod[b], pout.at[pl.ds(p_off(t), G)], sem_s[b]).wait()

    for b in range(2):
        start_gather(b, b)

    def pair(tt, carry):
        for b in range(2):
            t = 2 * tt + b
            wait_gather(t, b)

            @pl.when(tt >= 1)
            def _ws():
                wait_store(t - 2, b)

            def mulrow(j, c2):
                for k in range(D_HID // 16):
                    sl = pl.ds(k * 16, 16)
                    prod[b][j, sl] = rows_s[b][j, sl] * rows_d[b][j, sl]
                return c2

            lax.fori_loop(0, G, mulrow, 0, unroll=2)

            @pl.when(tt <= (NCHUNK - 2))
            def _ng():
                start_gather(t + 2, b)

            start_store(t, b)
        return carry

    lax.fori_loop(0, NCHUNK, pair, 0)
    for b in range(2):
        wait_store(NCHUNK2 - 2 + b, b)


_EB = 2560
_NEB = E // _EB


def _q_body(ea_lp_ref, p_lp_ref, ea_pl_ref, p_pl_ref, wc_lp_ref, wc_pl_ref,
            out_ref, q_lp, qb_lp, q_pl, qb_pl):
    i = pl.program_id(0)

    @pl.when(i == 0)
    def _init():
        q_lp[...] = jnp.zeros_like(q_lp)
        qb_lp[...] = jnp.zeros_like(qb_lp)
        q_pl[...] = jnp.zeros_like(q_pl)
        qb_pl[...] = jnp.zeros_like(qb_pl)

    q_lp[...] += jnp.dot(ea_lp_ref[...], p_lp_ref[...],
                         preferred_element_type=jnp.float32, precision=lax.Precision.HIGHEST)
    qb_lp[...] += jnp.sum(p_lp_ref[...], axis=0, keepdims=True)
    q_pl[...] += jnp.dot(ea_pl_ref[...], p_pl_ref[...],
                         preferred_element_type=jnp.float32, precision=lax.Precision.HIGHEST)
    qb_pl[...] += jnp.sum(p_pl_ref[...], axis=0, keepdims=True)

    @pl.when(i == _NEB - 1)
    def _fin():
        wc_lp = wc_lp_ref[...]
        wc_pl = wc_pl_ref[...]
        t_lp = (jnp.sum(wc_lp[:D_EDGE] * q_lp[...])
                + jnp.sum(wc_lp[D_EDGE:D_EDGE + 1] * qb_lp[...])
                + jnp.sum(wc_lp[D_EDGE + 1:D_EDGE + 2]))
        t_pl = (jnp.sum(wc_pl[:D_EDGE] * q_pl[...])
                + jnp.sum(wc_pl[D_EDGE:D_EDGE + 1] * qb_pl[...])
                + jnp.sum(wc_pl[D_EDGE + 1:D_EDGE + 2]))
        col = lax.broadcasted_iota(jnp.int32, (1, 2), 1)
        out_ref[...] = jnp.where(col == 0, t_lp, t_pl)


def _q_reduce(ea_t_lp, p_full, ea_t_pl, wc_lp, wc_pl):
    f = jnp.float32
    return pl.pallas_call(
        _q_body,
        grid=(_NEB,),
        in_specs=[
            pl.BlockSpec((D_EDGE, _EB), lambda i: (0, i)),
            pl.BlockSpec((_EB, D_HID), lambda i: (i, 0)),
            pl.BlockSpec((D_EDGE, _EB), lambda i: (0, i)),
            pl.BlockSpec((_EB, D_HID), lambda i: (i + _NEB, 0)),
            pl.BlockSpec((24, D_HID), lambda i: (0, 0)),
            pl.BlockSpec((24, D_HID), lambda i: (0, 0)),
        ],
        out_specs=pl.BlockSpec((1, 2), lambda i: (0, 0)),
        out_shape=jax.ShapeDtypeStruct((1, 2), f),
        scratch_shapes=[
            pltpu.VMEM((D_EDGE, D_HID), f),
            pltpu.VMEM((1, D_HID), f),
            pltpu.VMEM((D_EDGE, D_HID), f),
            pltpu.VMEM((1, D_HID), f),
        ],
    )(ea_t_lp, p_full, ea_t_pl, p_full, wc_lp, wc_pl)


def kernel(x_ligand, x_pocket, edge_attr_l2p, edge_attr_p2l,
           W_lp_src, b_lp_src, W_lp_dst, b_lp_dst, W_lp_edge, b_lp_edge,
           w_fc_lp, b_fc_lp,
           W_pl_src, b_pl_src, W_pl_dst, b_pl_dst, W_pl_edge, b_pl_edge,
           w_fc_pl, b_fc_pl,
           edge_index_l2p, edge_index_p2l):
    f = jnp.float32

    wl = jnp.concatenate([W_lp_src, W_pl_dst], axis=1)
    wp = jnp.concatenate([W_lp_dst, W_pl_src], axis=1)
    bl = jnp.concatenate([b_lp_src, b_pl_dst])[None, :]
    bp = jnp.concatenate([b_lp_dst, b_pl_src])[None, :]

    src_lp = edge_index_l2p[0].astype(jnp.int32).reshape(NW, NCHUNK, G)
    dst_lp = edge_index_l2p[1].astype(jnp.int32).reshape(NW, NCHUNK, G)
    src_pl = (edge_index_p2l[0].astype(jnp.int32) + N_NODE).reshape(NW, NCHUNK, G)
    dst_pl = (edge_index_p2l[1].astype(jnp.int32) + N_NODE).reshape(NW, NCHUNK, G)
    srcc = jnp.concatenate([src_lp, src_pl], axis=1)
    dstc = jnp.concatenate([dst_lp, dst_pl], axis=1)

    ea_t_lp = edge_attr_l2p.T
    ea_t_pl = edge_attr_p2l.T

    def wcomb(We, be, wf, bf):
        wfv = wf[:, 0]
        rows = [We * wfv[None, :],
                (be * wfv)[None, :],
                jnp.full((1, D_HID), bf[0] * (E / float(D_HID)), dtype=f),
                jnp.zeros((24 - D_EDGE - 2, D_HID), dtype=f)]
        return jnp.concatenate(rows, axis=0)

    wc_lp = wcomb(W_lp_edge, b_lp_edge, w_fc_lp, b_fc_lp)
    wc_pl = wcomb(W_pl_edge, b_pl_edge, w_fc_pl, b_fc_pl)

    hs_lp, hd_pl, hd_lp, hs_pl = _proj(x_ligand, x_pocket, wl, wp, bl, bp)
    hs_full = jnp.concatenate([hs_lp, hs_pl], axis=0)
    hd_full = jnp.concatenate([hd_lp, hd_pl], axis=0)

    p_full = _sc_gather_mul(hs_full, hd_full, srcc, dstc)

    return _q_reduce(ea_t_lp, p_full, ea_t_pl, wc_lp, wc_pl)
